# baseline (speedup 1.0000x reference)
import jax
import jax.numpy as jnp
from jax.experimental import pallas as pl

N = 10000
E = 160000
MAXDEG = 10


def kernel(feats, edge_index, W0, b0, Wl, bl, Wr, gamma, beta):
    src = edge_index[0]
    dst = edge_index[1]
    deg = jnp.bincount(dst, length=N)
    deg = jnp.clip(deg, 0, MAXDEG)
    h = jax.ops.segment_sum(feats[src], dst, num_segments=N)
    out = feats @ W0.T + b0
    for d in range(1, MAXDEG + 1):
        cand = h @ Wl[d - 1].T + bl[d - 1] + feats @ Wr[d - 1].T
        out = jnp.where((deg == d)[:, None], cand, out)
    mean = out.mean(axis=0)
    var = out.var(axis=0)
    out = (out - mean) / jnp.sqrt(var + 1e-5) * gamma + beta
    self_idx = jnp.arange(N)
    src_all = jnp.concatenate([src, self_idx])
    dst_all = jnp.concatenate([dst, self_idx])
    out = jax.ops.segment_max(out[src_all], dst_all, num_segments=N)
    return out
